# baseline (speedup 1.0000x reference)
import functools

import jax
import jax.numpy as jnp
from jax import lax
from jax.experimental import pallas as pl
from jax.experimental.pallas import tpu as pltpu
from jax.experimental.pallas import tpu_sc as plsc

D_MODEL = 32


@jax.jit
def _embed_impl(xt, weight):
    S, BT = xt.shape
    D = D_MODEL
    info = plsc.get_sparse_core_info()
    NC = info.num_cores
    NW = NC * info.num_subcores
    NBT = BT // 128
    BTW = NBT // NW
    NFT = D // 8
    NH = 2 * S
    HB = 256

    mesh = plsc.VectorSubcoreMesh(core_axis_name="c", subcore_axis_name="s")

    @functools.partial(
        pl.kernel,
        mesh=mesh,
        out_type=jax.ShapeDtypeStruct((S, NBT, 128), jnp.int32),
        scratch_types=[pltpu.VMEM((8, 128), jnp.int32)],
        compiler_params=pltpu.CompilerParams(use_tc_tiling_on_sc=True),
    )
    def untile(xt_hbm, idx_hbm, tile_v):
        wid = lax.axis_index("s") * NC + lax.axis_index("c")
        for c in range(BTW):
            bt = wid * BTW + c
            for r in range((S + 7) // 8):
                h = min(8, S - r * 8)
                pltpu.sync_copy(
                    xt_hbm.at[pl.ds(r * 8, h), pl.ds(bt * 128, 128)],
                    tile_v.at[pl.ds(0, h)],
                )
                pltpu.sync_copy(
                    tile_v.at[pl.ds(0, h)],
                    idx_hbm.at[pl.ds(r * 8, h), bt],
                )

    @functools.partial(
        pl.kernel,
        mesh=mesh,
        out_type=jax.ShapeDtypeStruct((S, NFT, NBT * 1024), jnp.float32),
        scratch_types=[
            pltpu.VMEM((S, BTW, 128), jnp.int32),
            pltpu.VMEM((HB,), jnp.int32),
            pltpu.VMEM((HB,), jnp.int32),
            pltpu.VMEM((HB, 128), jnp.float32),
            pltpu.VMEM((HB, 128), jnp.float32),
            pltpu.VMEM((NFT * 2048,), jnp.float32),
            pltpu.VMEM((NFT * 2048,), jnp.float32),
            pltpu.SemaphoreType.DMA,
            pltpu.SemaphoreType.DMA,
            pltpu.SemaphoreType.DMA,
            pltpu.SemaphoreType.DMA,
        ],
        compiler_params=pltpu.CompilerParams(
            use_tc_tiling_on_sc=False, needs_layout_passes=False
        ),
    )
    def gat(idx_hbm, tab_hbm, out_hbm, idx_all, qb0, qb1, blk0, blk1,
            t0, t1, gsem0, gsem1, osem0, osem1):
        wid = lax.axis_index("s") * NC + lax.axis_index("c")
        bt0 = wid * BTW
        iota = lax.iota(jnp.int32, 16)

        pltpu.sync_copy(idx_hbm.at[:, pl.ds(bt0, BTW)], idx_all)

        def load_idx_chunk(hs, m):
            s = hs >> 1
            bq = ((hs & 1) << 1) + (m >> 3)
            return idx_all[s, bq, pl.ds((m & 7) * 16, 16)]

        def compute_q(hs, qb):
            @plsc.parallel_loop(0, 16, unroll=4)
            def mbody(m):
                v = load_idx_chunk(hs, m)
                qb[pl.ds(m * 16, 16)] = lax.shift_right_logical(v, 2)

        def fire_gather(qb, blk, gsem):
            pltpu.async_copy(tab_hbm.at[qb], blk, gsem)

        def drain_gather(gsem):
            pltpu.make_async_copy(tab_hbm.at[qb0], blk0, gsem).wait()

        def transpose(hs, blk, t):
            @plsc.parallel_loop(0, 16, unroll=4)
            def mbody(m):
                v = load_idx_chunk(hs, m)
                r32 = lax.shift_left(v & 3, 5)
                rowv = iota + m * 16
                off = ((m >> 3) << 10) + ((m & 7) << 4)
                for ft in range(NFT):
                    for fs in range(8):
                        colv = r32 + (ft * 8 + fs)
                        val = plsc.load_gather(blk, [rowv, colv])
                        t[pl.ds(off + ft * 2048 + fs * 128, 16)] = val

        def fire_wb(hs, t, osem):
            s = hs >> 1
            o0 = bt0 * 1024 + (hs & 1) * 2048
            for ft in range(NFT):
                pltpu.async_copy(
                    t.at[pl.ds(ft * 2048, 2048)],
                    out_hbm.at[s, ft, pl.ds(o0, 2048)],
                    osem,
                )

        def drain_wb(osem):
            for _ in range(NFT):
                pltpu.make_async_copy(
                    t0.at[pl.ds(0, 2048)],
                    out_hbm.at[0, 0, pl.ds(0, 2048)],
                    osem,
                ).wait()

        compute_q(0, qb0)
        fire_gather(qb0, blk0, gsem0)
        compute_q(1, qb1)
        fire_gather(qb1, blk1, gsem1)

        def kbody(k, carry):
            hs0 = k * 2
            for p, (qb, blk, t, gsem, osem) in enumerate(
                ((qb0, blk0, t0, gsem0, osem0), (qb1, blk1, t1, gsem1, osem1))
            ):
                hs = hs0 + p
                drain_gather(gsem)

                @pl.when(k > 0)
                def _(osem=osem):
                    drain_wb(osem)

                transpose(hs, blk, t)
                fire_wb(hs, t, osem)

                @pl.when(hs + 2 < NH)
                def _(qb=qb, blk=blk, hs=hs, gsem=gsem):
                    compute_q(hs + 2, qb)
                    fire_gather(qb, blk, gsem)

            return carry

        lax.fori_loop(0, NH // 2, kbody, 0)
        drain_wb(osem0)
        drain_wb(osem1)

    idx3 = untile(xt)
    tab = weight.reshape(weight.shape[0] * D // 128, 128)
    out_lin = gat(idx3, tab)
    return out_lin


def kernel(x, weight):
    out_lin = _embed_impl(x.T, weight)
    B, S = x.shape
    return (
        out_lin.reshape(S, 4, 128, 8, 128)
        .transpose(2, 4, 0, 1, 3)
        .reshape(B, S, D_MODEL)
    )


# speedup vs baseline: 1.0307x; 1.0307x over previous
import functools

import jax
import jax.numpy as jnp
from jax import lax
from jax.experimental import pallas as pl
from jax.experimental.pallas import tpu as pltpu
from jax.experimental.pallas import tpu_sc as plsc

D_MODEL = 32


@jax.jit
def _embed_impl(xt, wt):
    S, BT = xt.shape
    D, V = wt.shape
    info = plsc.get_sparse_core_info()
    NC = info.num_cores
    NW = NC * info.num_subcores
    NBT = BT // 128
    BTW = NBT // NW
    NFT = D // 8
    NH = 2 * S
    HB = 256

    NVB = V // 128
    VTAIL = V - NVB * 128
    NBW = NVB // NW
    NREM = NVB - NBW * NW

    mesh = plsc.VectorSubcoreMesh(core_axis_name="c", subcore_axis_name="s")

    @functools.partial(
        pl.kernel,
        mesh=mesh,
        out_type=jax.ShapeDtypeStruct((S, NBT, 128), jnp.int32),
        scratch_types=[pltpu.VMEM((8, 128), jnp.int32)],
        compiler_params=pltpu.CompilerParams(use_tc_tiling_on_sc=True),
    )
    def untile(xt_hbm, idx_hbm, tile_v):
        wid = lax.axis_index("s") * NC + lax.axis_index("c")
        for c in range(BTW):
            bt = wid * BTW + c
            for r in range((S + 7) // 8):
                h = min(8, S - r * 8)
                pltpu.sync_copy(
                    xt_hbm.at[pl.ds(r * 8, h), pl.ds(bt * 128, 128)],
                    tile_v.at[pl.ds(0, h)],
                )
                pltpu.sync_copy(
                    tile_v.at[pl.ds(0, h)],
                    idx_hbm.at[pl.ds(r * 8, h), bt],
                )

    @functools.partial(
        pl.kernel,
        mesh=mesh,
        out_type=jax.ShapeDtypeStruct((V * D,), jnp.float32),
        scratch_types=[
            pltpu.VMEM((D, 128), jnp.float32),
            pltpu.VMEM((D, 128), jnp.float32),
            pltpu.VMEM((128 * D,), jnp.float32),
            pltpu.VMEM((128 * D,), jnp.float32),
            pltpu.VMEM((D, VTAIL), jnp.float32),
            pltpu.SemaphoreType.DMA,
            pltpu.SemaphoreType.DMA,
            pltpu.SemaphoreType.DMA,
            pltpu.SemaphoreType.DMA,
        ],
        compiler_params=pltpu.CompilerParams(
            use_tc_tiling_on_sc=True, needs_layout_passes=False
        ),
    )
    def retile(wt_hbm, wf_hbm, in0, in1, st0, st1, in_t, is0, is1, os0, os1):
        wid = lax.axis_index("s") * NC + lax.axis_index("c")
        c0 = wid * NBW
        iota = lax.iota(jnp.int32, 16)

        def fire_in(j, ibuf, isem):
            pltpu.async_copy(
                wt_hbm.at[:, pl.ds((c0 + j) * 128, 128)], ibuf, isem
            )

        def drain_in(ibuf, isem):
            pltpu.make_async_copy(
                wt_hbm.at[:, pl.ds(0, 128)], ibuf, isem
            ).wait()

        def transpose_block(ibuf, st, width):
            @plsc.parallel_loop(0, width * 2, unroll=4)
            def tbody(k2):
                rowv = iota + ((k2 & 1) << 4)
                colv = jnp.full((16,), 0, jnp.int32) + (k2 >> 1)
                val = plsc.load_gather(ibuf, [rowv, colv])
                st[pl.ds(k2 * 16, 16)] = val

        def fire_out(j, st, osem):
            pltpu.async_copy(
                st, wf_hbm.at[pl.ds((c0 + j) * 128 * D, 128 * D)], osem
            )

        def drain_out(st, osem):
            pltpu.make_async_copy(
                st, wf_hbm.at[pl.ds(0, 128 * D)], osem
            ).wait()

        fire_in(0, in0, is0)
        fire_in(1, in1, is1)

        def kbody(k, carry):
            j0 = k * 2
            for p, (ibuf, st, isem, osem) in enumerate(
                ((in0, st0, is0, os0), (in1, st1, is1, os1))
            ):
                j = j0 + p
                drain_in(ibuf, isem)

                @pl.when(k > 0)
                def _(st=st, osem=osem):
                    drain_out(st, osem)

                transpose_block(ibuf, st, 128)
                fire_out(j, st, osem)

                @pl.when(j + 2 < NBW)
                def _(j=j, ibuf=ibuf, isem=isem):
                    fire_in(j + 2, ibuf, isem)

            return carry

        lax.fori_loop(0, NBW // 2, kbody, 0)
        drain_out(st0, os0)
        drain_out(st1, os1)

        @pl.when(wid < NREM)
        def _():
            c = NVB - NREM + wid
            pltpu.sync_copy(wt_hbm.at[:, pl.ds(c * 128, 128)], in0)
            transpose_block(in0, st0, 128)
            pltpu.sync_copy(st0, wf_hbm.at[pl.ds(c * 128 * D, 128 * D)])

        @pl.when(wid == NREM)
        def _():
            pltpu.sync_copy(wt_hbm.at[:, pl.ds(NVB * 128, VTAIL)], in_t)
            transpose_block(in_t, st0, VTAIL)
            pltpu.sync_copy(
                st0.at[pl.ds(0, VTAIL * D)],
                wf_hbm.at[pl.ds(NVB * 128 * D, VTAIL * D)],
            )

    @functools.partial(
        pl.kernel,
        mesh=mesh,
        out_type=jax.ShapeDtypeStruct((S, NFT, NBT * 1024), jnp.float32),
        scratch_types=[
            pltpu.VMEM((S, BTW, 128), jnp.int32),
            pltpu.VMEM((HB, D), jnp.float32),
            pltpu.VMEM((HB, D), jnp.float32),
            pltpu.VMEM((NFT * 2048,), jnp.float32),
            pltpu.VMEM((NFT * 2048,), jnp.float32),
            pltpu.SemaphoreType.DMA,
            pltpu.SemaphoreType.DMA,
            pltpu.SemaphoreType.DMA,
            pltpu.SemaphoreType.DMA,
        ],
        compiler_params=pltpu.CompilerParams(
            use_tc_tiling_on_sc=False, needs_layout_passes=False
        ),
    )
    def gat(idx_hbm, tab_hbm, out_hbm, idx_all, blk0, blk1,
            t0, t1, gsem0, gsem1, osem0, osem1):
        wid = lax.axis_index("s") * NC + lax.axis_index("c")
        bt0 = wid * BTW
        iota = lax.iota(jnp.int32, 16)

        pltpu.sync_copy(idx_hbm.at[:, pl.ds(bt0, BTW)], idx_all)

        def fire_gather(hs, blk, gsem):
            s = hs >> 1
            h = hs & 1
            for q in range(2):
                pltpu.async_copy(
                    tab_hbm.at[idx_all.at[s, h * 2 + q]],
                    blk.at[pl.ds(q * 128, 128)],
                    gsem,
                )

        def drain_gather(gsem):
            for _ in range(2):
                pltpu.make_async_copy(
                    tab_hbm.at[idx_all.at[0, 0]],
                    blk0.at[pl.ds(0, 128)],
                    gsem,
                ).wait()

        def transpose(blk, t):
            @plsc.parallel_loop(0, 16, unroll=2)
            def mbody(m):
                rowv = iota + m * 16
                off = ((m >> 3) << 10) + ((m & 7) << 4)
                for ft in range(NFT):
                    for fs in range(8):
                        colv = jnp.full((16,), ft * 8 + fs, jnp.int32)
                        val = plsc.load_gather(blk, [rowv, colv])
                        t[pl.ds(off + ft * 2048 + fs * 128, 16)] = val

        def fire_wb(hs, t, osem):
            s = hs >> 1
            o0 = bt0 * 1024 + (hs & 1) * 2048
            for ft in range(NFT):
                pltpu.async_copy(
                    t.at[pl.ds(ft * 2048, 2048)],
                    out_hbm.at[s, ft, pl.ds(o0, 2048)],
                    osem,
                )

        def drain_wb(osem):
            for _ in range(NFT):
                pltpu.make_async_copy(
                    t0.at[pl.ds(0, 2048)],
                    out_hbm.at[0, 0, pl.ds(0, 2048)],
                    osem,
                ).wait()

        fire_gather(0, blk0, gsem0)
        fire_gather(1, blk1, gsem1)

        def kbody(k, carry):
            hs0 = k * 2
            for p, (blk, t, gsem, osem) in enumerate(
                ((blk0, t0, gsem0, osem0), (blk1, t1, gsem1, osem1))
            ):
                hs = hs0 + p
                drain_gather(gsem)

                @pl.when(k > 0)
                def _(osem=osem):
                    drain_wb(osem)

                transpose(blk, t)
                fire_wb(hs, t, osem)

                @pl.when(hs + 2 < NH)
                def _(hs=hs, blk=blk, gsem=gsem):
                    fire_gather(hs + 2, blk, gsem)

            return carry

        lax.fori_loop(0, NH // 2, kbody, 0)
        drain_wb(osem0)
        drain_wb(osem1)

    idx3 = untile(xt)
    wflat = retile(wt)
    tab = wflat.reshape(V, D)
    out_lin = gat(idx3, tab)
    return out_lin


def kernel(x, weight):
    out_lin = _embed_impl(x.T, weight.T)
    B, S = x.shape
    return (
        out_lin.reshape(S, 4, 128, 8, 128)
        .transpose(2, 4, 0, 1, 3)
        .reshape(B, S, D_MODEL)
    )


# speedup vs baseline: 1.0307x; 1.0000x over previous
import functools

import jax
import jax.numpy as jnp
from jax import lax
from jax.experimental import pallas as pl
from jax.experimental.pallas import tpu as pltpu
from jax.experimental.pallas import tpu_sc as plsc

D_MODEL = 32


@jax.jit
def _embed_impl(xt, wt):
    S, BT = xt.shape
    D, V = wt.shape
    info = plsc.get_sparse_core_info()
    NC = info.num_cores
    NW = NC * info.num_subcores
    NBT = BT // 128
    BTW = NBT // NW
    NFT = D // 8
    NH = 2 * S
    HB = 256

    NVB = V // 128
    VTAIL = V - NVB * 128
    NBW = NVB // NW
    NREM = NVB - NBW * NW

    mesh = plsc.VectorSubcoreMesh(core_axis_name="c", subcore_axis_name="s")

    @functools.partial(
        pl.kernel,
        mesh=mesh,
        out_type=jax.ShapeDtypeStruct((S, NBT, 128), jnp.int32),
        scratch_types=[pltpu.VMEM((8, 128), jnp.int32)],
        compiler_params=pltpu.CompilerParams(use_tc_tiling_on_sc=True),
    )
    def untile(xt_hbm, idx_hbm, tile_v):
        wid = lax.axis_index("s") * NC + lax.axis_index("c")
        for c in range(BTW):
            bt = wid * BTW + c
            for r in range((S + 7) // 8):
                h = min(8, S - r * 8)
                pltpu.sync_copy(
                    xt_hbm.at[pl.ds(r * 8, h), pl.ds(bt * 128, 128)],
                    tile_v.at[pl.ds(0, h)],
                )
                pltpu.sync_copy(
                    tile_v.at[pl.ds(0, h)],
                    idx_hbm.at[pl.ds(r * 8, h), bt],
                )

    @functools.partial(
        pl.kernel,
        mesh=mesh,
        out_type=jax.ShapeDtypeStruct((V * D,), jnp.float32),
        scratch_types=[
            pltpu.VMEM((D, 128), jnp.float32),
            pltpu.VMEM((D, 128), jnp.float32),
            pltpu.VMEM((128 * D,), jnp.float32),
            pltpu.VMEM((128 * D,), jnp.float32),
            pltpu.VMEM((D, VTAIL), jnp.float32),
            pltpu.SemaphoreType.DMA,
            pltpu.SemaphoreType.DMA,
            pltpu.SemaphoreType.DMA,
            pltpu.SemaphoreType.DMA,
        ],
        compiler_params=pltpu.CompilerParams(
            use_tc_tiling_on_sc=True, needs_layout_passes=False,
            disable_bounds_checks=True,
        ),
    )
    def retile(wt_hbm, wf_hbm, in0, in1, st0, st1, in_t, is0, is1, os0, os1):
        wid = lax.axis_index("s") * NC + lax.axis_index("c")
        c0 = wid * NBW
        iota = lax.iota(jnp.int32, 16)

        def fire_in(j, ibuf, isem):
            pltpu.async_copy(
                wt_hbm.at[:, pl.ds((c0 + j) * 128, 128)], ibuf, isem
            )

        def drain_in(ibuf, isem):
            pltpu.make_async_copy(
                wt_hbm.at[:, pl.ds(0, 128)], ibuf, isem
            ).wait()

        def transpose_block(ibuf, st, width):
            @plsc.parallel_loop(0, width * 2, unroll=4)
            def tbody(k2):
                rowv = iota + ((k2 & 1) << 4)
                colv = jnp.full((16,), 0, jnp.int32) + (k2 >> 1)
                val = plsc.load_gather(ibuf, [rowv, colv])
                st[pl.ds(k2 * 16, 16)] = val

        def fire_out(j, st, osem):
            pltpu.async_copy(
                st, wf_hbm.at[pl.ds((c0 + j) * 128 * D, 128 * D)], osem
            )

        def drain_out(st, osem):
            pltpu.make_async_copy(
                st, wf_hbm.at[pl.ds(0, 128 * D)], osem
            ).wait()

        fire_in(0, in0, is0)
        fire_in(1, in1, is1)

        def kbody(k, carry):
            j0 = k * 2
            for p, (ibuf, st, isem, osem) in enumerate(
                ((in0, st0, is0, os0), (in1, st1, is1, os1))
            ):
                j = j0 + p
                drain_in(ibuf, isem)

                @pl.when(k > 0)
                def _(st=st, osem=osem):
                    drain_out(st, osem)

                transpose_block(ibuf, st, 128)
                fire_out(j, st, osem)

                @pl.when(j + 2 < NBW)
                def _(j=j, ibuf=ibuf, isem=isem):
                    fire_in(j + 2, ibuf, isem)

            return carry

        lax.fori_loop(0, NBW // 2, kbody, 0)
        drain_out(st0, os0)
        drain_out(st1, os1)

        @pl.when(wid < NREM)
        def _():
            c = NVB - NREM + wid
            pltpu.sync_copy(wt_hbm.at[:, pl.ds(c * 128, 128)], in0)
            transpose_block(in0, st0, 128)
            pltpu.sync_copy(st0, wf_hbm.at[pl.ds(c * 128 * D, 128 * D)])

        @pl.when(wid == NREM)
        def _():
            pltpu.sync_copy(wt_hbm.at[:, pl.ds(NVB * 128, VTAIL)], in_t)
            transpose_block(in_t, st0, VTAIL)
            pltpu.sync_copy(
                st0.at[pl.ds(0, VTAIL * D)],
                wf_hbm.at[pl.ds(NVB * 128 * D, VTAIL * D)],
            )

    @functools.partial(
        pl.kernel,
        mesh=mesh,
        out_type=jax.ShapeDtypeStruct((S, NFT, NBT * 1024), jnp.float32),
        scratch_types=[
            pltpu.VMEM((S, BTW, 128), jnp.int32),
            pltpu.VMEM((HB, D), jnp.float32),
            pltpu.VMEM((HB, D), jnp.float32),
            pltpu.VMEM((NFT * 2048,), jnp.float32),
            pltpu.VMEM((NFT * 2048,), jnp.float32),
            pltpu.SemaphoreType.DMA,
            pltpu.SemaphoreType.DMA,
            pltpu.SemaphoreType.DMA,
            pltpu.SemaphoreType.DMA,
        ],
        compiler_params=pltpu.CompilerParams(
            use_tc_tiling_on_sc=False, needs_layout_passes=False,
            disable_bounds_checks=True,
        ),
    )
    def gat(idx_hbm, tab_hbm, out_hbm, idx_all, blk0, blk1,
            t0, t1, gsem0, gsem1, osem0, osem1):
        wid = lax.axis_index("s") * NC + lax.axis_index("c")
        bt0 = wid * BTW
        iota = lax.iota(jnp.int32, 16)

        pltpu.sync_copy(idx_hbm.at[:, pl.ds(bt0, BTW)], idx_all)

        def fire_gather(hs, blk, gsem):
            s = hs >> 1
            h = hs & 1
            for q in range(2):
                pltpu.async_copy(
                    tab_hbm.at[idx_all.at[s, h * 2 + q]],
                    blk.at[pl.ds(q * 128, 128)],
                    gsem,
                )

        def drain_gather(gsem):
            for _ in range(2):
                pltpu.make_async_copy(
                    tab_hbm.at[idx_all.at[0, 0]],
                    blk0.at[pl.ds(0, 128)],
                    gsem,
                ).wait()

        def transpose(blk, t):
            @plsc.parallel_loop(0, 16, unroll=2)
            def mbody(m):
                rowv = iota + m * 16
                off = ((m >> 3) << 10) + ((m & 7) << 4)
                for ft in range(NFT):
                    for fs in range(8):
                        colv = jnp.full((16,), ft * 8 + fs, jnp.int32)
                        val = plsc.load_gather(blk, [rowv, colv])
                        t[pl.ds(off + ft * 2048 + fs * 128, 16)] = val

        def fire_wb(hs, t, osem):
            s = hs >> 1
            o0 = bt0 * 1024 + (hs & 1) * 2048
            for ft in range(NFT):
                pltpu.async_copy(
                    t.at[pl.ds(ft * 2048, 2048)],
                    out_hbm.at[s, ft, pl.ds(o0, 2048)],
                    osem,
                )

        def drain_wb(osem):
            for _ in range(NFT):
                pltpu.make_async_copy(
                    t0.at[pl.ds(0, 2048)],
                    out_hbm.at[0, 0, pl.ds(0, 2048)],
                    osem,
                ).wait()

        fire_gather(0, blk0, gsem0)
        fire_gather(1, blk1, gsem1)

        def kbody(k, carry):
            hs0 = k * 2
            for p, (blk, t, gsem, osem) in enumerate(
                ((blk0, t0, gsem0, osem0), (blk1, t1, gsem1, osem1))
            ):
                hs = hs0 + p
                drain_gather(gsem)

                @pl.when(k > 0)
                def _(osem=osem):
                    drain_wb(osem)

                transpose(blk, t)
                fire_wb(hs, t, osem)

                @pl.when(hs + 2 < NH)
                def _(hs=hs, blk=blk, gsem=gsem):
                    fire_gather(hs + 2, blk, gsem)

            return carry

        lax.fori_loop(0, NH // 2, kbody, 0)
        drain_wb(osem0)
        drain_wb(osem1)

    idx3 = untile(xt)
    wflat = retile(wt)
    tab = wflat.reshape(V, D)
    out_lin = gat(idx3, tab)
    return out_lin


def kernel(x, weight):
    out_lin = _embed_impl(x.T, weight.T)
    B, S = x.shape
    return (
        out_lin.reshape(S, 4, 128, 8, 128)
        .transpose(2, 4, 0, 1, 3)
        .reshape(B, S, D_MODEL)
    )


# speedup vs baseline: 1.2874x; 1.2491x over previous
import functools

import jax
import jax.numpy as jnp
from jax import lax
from jax.experimental import pallas as pl
from jax.experimental.pallas import tpu as pltpu
from jax.experimental.pallas import tpu_sc as plsc

D_MODEL = 32


@jax.jit
def _embed_impl(xt, wt):
    S, BT = xt.shape
    D, V = wt.shape
    info = plsc.get_sparse_core_info()
    NC = info.num_cores
    NW = NC * info.num_subcores
    NBT = BT // 128
    BTW = NBT // NW
    NFT = D // 8
    NH = 2 * S
    HB = 256

    NVB = V // 128
    VTAIL = V - NVB * 128
    NBW = NVB // NW
    NREM = NVB - NBW * NW

    mesh = plsc.VectorSubcoreMesh(core_axis_name="c", subcore_axis_name="s")

    @functools.partial(
        pl.kernel,
        mesh=mesh,
        out_type=jax.ShapeDtypeStruct((S, NBT, 128), jnp.int32),
        scratch_types=[pltpu.VMEM((8, 128), jnp.int32)],
        compiler_params=pltpu.CompilerParams(use_tc_tiling_on_sc=True),
    )
    def untile(xt_hbm, idx_hbm, tile_v):
        wid = lax.axis_index("s") * NC + lax.axis_index("c")
        for c in range(BTW):
            bt = wid * BTW + c
            for r in range((S + 7) // 8):
                h = min(8, S - r * 8)
                pltpu.sync_copy(
                    xt_hbm.at[pl.ds(r * 8, h), pl.ds(bt * 128, 128)],
                    tile_v.at[pl.ds(0, h)],
                )
                pltpu.sync_copy(
                    tile_v.at[pl.ds(0, h)],
                    idx_hbm.at[pl.ds(r * 8, h), bt],
                )

    @functools.partial(
        pl.kernel,
        mesh=mesh,
        out_type=jax.ShapeDtypeStruct((V * D,), jnp.float32),
        scratch_types=[
            pltpu.VMEM((D, 128), jnp.float32),
            pltpu.VMEM((D, 128), jnp.float32),
            pltpu.VMEM((128 * D,), jnp.float32),
            pltpu.VMEM((128 * D,), jnp.float32),
            pltpu.VMEM((D, VTAIL), jnp.float32),
            pltpu.SemaphoreType.DMA,
            pltpu.SemaphoreType.DMA,
            pltpu.SemaphoreType.DMA,
            pltpu.SemaphoreType.DMA,
        ],
        compiler_params=pltpu.CompilerParams(
            use_tc_tiling_on_sc=True, needs_layout_passes=False,
            disable_bounds_checks=True,
        ),
    )
    def retile(wt_hbm, wf_hbm, in0, in1, st0, st1, in_t, is0, is1, os0, os1):
        wid = lax.axis_index("s") * NC + lax.axis_index("c")
        c0 = wid * NBW
        iota = lax.iota(jnp.int32, 16)

        def fire_in(j, ibuf, isem):
            pltpu.async_copy(
                wt_hbm.at[:, pl.ds((c0 + j) * 128, 128)], ibuf, isem
            )

        def drain_in(ibuf, isem):
            pltpu.make_async_copy(
                wt_hbm.at[:, pl.ds(0, 128)], ibuf, isem
            ).wait()

        def transpose_block(ibuf, st, width):
            @plsc.parallel_loop(0, width * 2, unroll=4)
            def tbody(k2):
                rowv = iota + ((k2 & 1) << 4)
                colv = jnp.full((16,), 0, jnp.int32) + (k2 >> 1)
                val = plsc.load_gather(ibuf, [rowv, colv])
                st[pl.ds(k2 * 16, 16)] = val

        def fire_out(j, st, osem):
            pltpu.async_copy(
                st, wf_hbm.at[pl.ds((c0 + j) * 128 * D, 128 * D)], osem
            )

        def drain_out(st, osem):
            pltpu.make_async_copy(
                st, wf_hbm.at[pl.ds(0, 128 * D)], osem
            ).wait()

        fire_in(0, in0, is0)
        fire_in(1, in1, is1)

        def kbody(k, carry):
            j0 = k * 2
            for p, (ibuf, st, isem, osem) in enumerate(
                ((in0, st0, is0, os0), (in1, st1, is1, os1))
            ):
                j = j0 + p
                drain_in(ibuf, isem)

                @pl.when(k > 0)
                def _(st=st, osem=osem):
                    drain_out(st, osem)

                transpose_block(ibuf, st, 128)
                fire_out(j, st, osem)

                @pl.when(j + 2 < NBW)
                def _(j=j, ibuf=ibuf, isem=isem):
                    fire_in(j + 2, ibuf, isem)

            return carry

        lax.fori_loop(0, NBW // 2, kbody, 0)
        drain_out(st0, os0)
        drain_out(st1, os1)

        @pl.when(wid < NREM)
        def _():
            c = NVB - NREM + wid
            pltpu.sync_copy(wt_hbm.at[:, pl.ds(c * 128, 128)], in0)
            transpose_block(in0, st0, 128)
            pltpu.sync_copy(st0, wf_hbm.at[pl.ds(c * 128 * D, 128 * D)])

        @pl.when(wid == NREM)
        def _():
            pltpu.sync_copy(wt_hbm.at[:, pl.ds(NVB * 128, VTAIL)], in_t)
            transpose_block(in_t, st0, VTAIL)
            pltpu.sync_copy(
                st0.at[pl.ds(0, VTAIL * D)],
                wf_hbm.at[pl.ds(NVB * 128 * D, VTAIL * D)],
            )

    @functools.partial(
        pl.kernel,
        mesh=mesh,
        out_type=jax.ShapeDtypeStruct((S, NFT, NBT * 8, 128), jnp.float32),
        scratch_types=[
            pltpu.VMEM((S, BTW, 128), jnp.int32),
            pltpu.VMEM((HB, D), jnp.float32),
            pltpu.VMEM((HB, D), jnp.float32),
            pltpu.VMEM((64, 137), jnp.float32),
            pltpu.VMEM((64, 137), jnp.float32),
            pltpu.SemaphoreType.DMA,
            pltpu.SemaphoreType.DMA,
            pltpu.SemaphoreType.DMA,
            pltpu.SemaphoreType.DMA,
        ],
        compiler_params=pltpu.CompilerParams(
            use_tc_tiling_on_sc=False, needs_layout_passes=False,
            disable_bounds_checks=True,
        ),
    )
    def gat(idx_hbm, tab_hbm, out_hbm, idx_all, blk0, blk1,
            t0, t1, gsem0, gsem1, osem0, osem1):
        wid = lax.axis_index("s") * NC + lax.axis_index("c")
        bt0 = wid * BTW
        iota = lax.iota(jnp.int32, 16)

        pltpu.sync_copy(idx_hbm.at[:, pl.ds(bt0, BTW)], idx_all)

        def fire_gather(hs, blk, gsem):
            s = hs >> 1
            h = hs & 1
            for q in range(2):
                pltpu.async_copy(
                    tab_hbm.at[idx_all.at[s, h * 2 + q]],
                    blk.at[pl.ds(q * 128, 128)],
                    gsem,
                )

        def drain_gather(gsem):
            for _ in range(2):
                pltpu.make_async_copy(
                    tab_hbm.at[idx_all.at[0, 0]],
                    blk0.at[pl.ds(0, 128)],
                    gsem,
                ).wait()

        r0c = lax.shift_right_logical(iota, 3) * 16 + (iota & 7)

        def transpose(blk, t):
            @plsc.parallel_loop(0, HB, unroll=2)
            def bbody(b):
                btl = lax.shift_right_logical(b, 7)
                bs = b & 127
                v0 = blk[b, pl.ds(0, 16)]
                v1 = blk[b, pl.ds(16, 16)]
                row0 = r0c + btl * 8
                col = jnp.full((16,), 0, jnp.int32) + bs
                plsc.store_scatter(t, [row0, col], v0)
                plsc.store_scatter(t, [row0 + 32, col], v1)

        def fire_wb(hs, t, osem):
            s = hs >> 1
            r0 = (bt0 + (hs & 1) * 2) * 8
            for ft in range(NFT):
                pltpu.async_copy(
                    t.at[pl.ds(ft * 16, 16), pl.ds(0, 128)],
                    out_hbm.at[s, ft, pl.ds(r0, 16)],
                    osem,
                )

        def drain_wb(osem):
            for _ in range(NFT):
                pltpu.make_async_copy(
                    t0.at[pl.ds(0, 16), pl.ds(0, 128)],
                    out_hbm.at[0, 0, pl.ds(0, 16)],
                    osem,
                ).wait()

        fire_gather(0, blk0, gsem0)
        fire_gather(1, blk1, gsem1)

        def kbody(k, carry):
            hs0 = k * 2
            for p, (blk, t, gsem, osem) in enumerate(
                ((blk0, t0, gsem0, osem0), (blk1, t1, gsem1, osem1))
            ):
                hs = hs0 + p
                drain_gather(gsem)

                @pl.when(k > 0)
                def _(osem=osem):
                    drain_wb(osem)

                transpose(blk, t)
                fire_wb(hs, t, osem)

                @pl.when(hs + 2 < NH)
                def _(hs=hs, blk=blk, gsem=gsem):
                    fire_gather(hs + 2, blk, gsem)

            return carry

        lax.fori_loop(0, NH // 2, kbody, 0)
        drain_wb(osem0)
        drain_wb(osem1)

    idx3 = untile(xt)
    wflat = retile(wt)
    tab = wflat.reshape(V, D)
    out_lin = gat(idx3, tab)
    return out_lin


def kernel(x, weight):
    out_lin = _embed_impl(x.T, weight.T)
    B, S = x.shape
    return (
        out_lin.reshape(S, 4, 128, 8, 128)
        .transpose(2, 4, 0, 1, 3)
        .reshape(B, S, D_MODEL)
    )
